# Initial kernel scaffold; baseline (speedup 1.0000x reference)
import jax
import jax.numpy as jnp
from jax.experimental import pallas as pl


def kernel(x, edge_index, W0, att_src0, att_dst0, bias0, gamma0, beta0, W1, att_src1, att_dst1, bias1, gamma1, beta1):
    raise NotImplementedError("write your pallas kernel here")


# speedup vs baseline: 1.1357x; 1.1357x over previous
import functools

import jax
import jax.numpy as jnp
from jax.experimental import pallas as pl
from jax.experimental.pallas import tpu as pltpu


def _ln_relu_kernel(x_ref, g_ref, b_ref, o_ref, *, relu):
    x = x_ref[...]
    mu = jnp.mean(x, axis=-1, keepdims=True)
    var = jnp.mean((x - mu) ** 2, axis=-1, keepdims=True)
    y = (x - mu) * jax.lax.rsqrt(var + 1e-5) * g_ref[...] + b_ref[...]
    if relu:
        y = jnp.maximum(y, 0.0)
    o_ref[...] = y


def _ln(x, gamma, beta, relu):
    n, d = x.shape
    blk = 1000
    return pl.pallas_call(
        functools.partial(_ln_relu_kernel, relu=relu),
        grid=(n // blk,),
        in_specs=[
            pl.BlockSpec((blk, d), lambda i: (i, 0)),
            pl.BlockSpec((1, d), lambda i: (0, 0)),
            pl.BlockSpec((1, d), lambda i: (0, 0)),
        ],
        out_specs=pl.BlockSpec((blk, d), lambda i: (i, 0)),
        out_shape=jax.ShapeDtypeStruct((n, d), jnp.float32),
    )(x, gamma.reshape(1, d), beta.reshape(1, d))


def _gat_layer(x, edge_index, W, att_src, att_dst, bias, heads, out_dim, concat):
    N = x.shape[0]
    h = (x @ W).reshape(N, heads, out_dim)
    src = edge_index[0]
    dst = edge_index[1]
    a_src = jnp.sum(h * att_src[None, :, :], axis=-1)
    a_dst = jnp.sum(h * att_dst[None, :, :], axis=-1)
    e = a_src[src] + a_dst[dst]
    e = jax.nn.leaky_relu(e, negative_slope=0.2)
    ex = jnp.exp(e)
    denom = jax.ops.segment_sum(ex, dst, num_segments=N)
    msg = h[src] * ex[:, :, None]
    out = jax.ops.segment_sum(msg, dst, num_segments=N)
    out = out / (denom[:, :, None] + 1e-16)
    if concat:
        out = out.reshape(N, heads * out_dim)
    else:
        out = jnp.mean(out, axis=1)
    return out + bias


def kernel(x, edge_index, W0, att_src0, att_dst0, bias0, gamma0, beta0,
           W1, att_src1, att_dst1, bias1, gamma1, beta1):
    h = _gat_layer(x, edge_index, W0, att_src0, att_dst0, bias0, 8, 128, True)
    h = _ln(h, gamma0, beta0, relu=True)
    out = _gat_layer(h, edge_index, W1, att_src1, att_dst1, bias1, 1, 128, False)
    out = _ln(out, gamma1, beta1, relu=False)
    return out


# speedup vs baseline: 9.7766x; 8.6084x over previous
import functools

import jax
import jax.numpy as jnp
from jax import lax
from jax.experimental import pallas as pl
from jax.experimental.pallas import tpu as pltpu
from jax.experimental.pallas import tpu_sc as plsc

N = 10000
NP = 10240
E = 320000
NC = 2
NS = 16
LANES = 16
CA = 80
CB = 80
ROWS_PER_SUB = NP // NS
ZROWS = 128

_f32 = jnp.float32
_i32 = jnp.int32


def _mm0_kernel(x_ref, w_ref, o_ref):
    o_ref[0] = jnp.dot(x_ref[...], w_ref[...], preferred_element_type=_f32)


def _mm0(x, W0):
    R = 1000
    return pl.pallas_call(
        _mm0_kernel,
        grid=(8, N // R),
        in_specs=[
            pl.BlockSpec((R, 128), lambda h, i: (i, 0)),
            pl.BlockSpec((128, 128), lambda h, i: (0, h)),
        ],
        out_specs=pl.BlockSpec((1, R, 128), lambda h, i: (h, i, 0)),
        out_shape=jax.ShapeDtypeStruct((8, N, 128), _f32),
    )(x, W0)


def _amm0_kernel(hT_ref, asrc_ref, adst_ref, osrc_ref, odst_ref):
    h = pl.program_id(1)
    onehot = (lax.broadcasted_iota(_i32, (1, 16), 1) == h).astype(_f32)
    rs = jnp.sum(hT_ref[0] * asrc_ref[0], axis=1, keepdims=True)
    rd = jnp.sum(hT_ref[0] * adst_ref[0], axis=1, keepdims=True)

    @pl.when(h == 0)
    def _():
        osrc_ref[...] = jnp.zeros_like(osrc_ref)
        odst_ref[...] = jnp.zeros_like(odst_ref)

    osrc_ref[...] += rs * onehot
    odst_ref[...] += rd * onehot


def _amm0(hT, att_src0, att_dst0):
    R = 1000
    return pl.pallas_call(
        _amm0_kernel,
        grid=(N // R, 8),
        in_specs=[
            pl.BlockSpec((1, R, 128), lambda i, h: (h, i, 0)),
            pl.BlockSpec((1, 1, 128), lambda i, h: (h, 0, 0)),
            pl.BlockSpec((1, 1, 128), lambda i, h: (h, 0, 0)),
        ],
        out_specs=[
            pl.BlockSpec((R, 16), lambda i, h: (i, 0)),
            pl.BlockSpec((R, 16), lambda i, h: (i, 0)),
        ],
        out_shape=[
            jax.ShapeDtypeStruct((NP, 16), _f32),
            jax.ShapeDtypeStruct((NP, 16), _f32),
        ],
    )(hT, att_src0.reshape(8, 1, 128), att_dst0.reshape(8, 1, 128))


def _mm1_kernel(y_ref, w_ref, o_ref):
    h = pl.program_id(1)
    acc = jnp.dot(y_ref[0], w_ref[0], preferred_element_type=_f32)

    @pl.when(h == 0)
    def _():
        o_ref[...] = acc

    @pl.when(h != 0)
    def _():
        o_ref[...] += acc


def _mm1(yT, W1r):
    R = 1000
    return pl.pallas_call(
        _mm1_kernel,
        grid=(N // R, 8),
        in_specs=[
            pl.BlockSpec((1, R, 128), lambda i, h: (h, i, 0)),
            pl.BlockSpec((1, 128, 128), lambda i, h: (h, 0, 0)),
        ],
        out_specs=pl.BlockSpec((R, 128), lambda i, h: (i, 0)),
        out_shape=jax.ShapeDtypeStruct((N, 128), _f32),
    )(yT, W1r)


def _amm1_kernel(h1_ref, asrc_ref, adst_ref, osrc_ref, odst_ref):
    onehot0 = (lax.broadcasted_iota(_i32, (1, 16), 1) == 0).astype(_f32)
    rs = jnp.sum(h1_ref[...] * asrc_ref[...], axis=1, keepdims=True)
    rd = jnp.sum(h1_ref[...] * adst_ref[...], axis=1, keepdims=True)
    osrc_ref[...] = rs * onehot0
    odst_ref[...] = rd * onehot0


def _amm1(h1, att_src1, att_dst1):
    R = 1000
    return pl.pallas_call(
        _amm1_kernel,
        grid=(N // R,),
        in_specs=[
            pl.BlockSpec((R, 128), lambda i: (i, 0)),
            pl.BlockSpec((1, 128), lambda i: (0, 0)),
            pl.BlockSpec((1, 128), lambda i: (0, 0)),
        ],
        out_specs=[
            pl.BlockSpec((R, 16), lambda i: (i, 0)),
            pl.BlockSpec((R, 16), lambda i: (i, 0)),
        ],
        out_shape=[
            jax.ShapeDtypeStruct((NP, 16), _f32),
            jax.ShapeDtypeStruct((NP, 16), _f32),
        ],
    )(h1, att_src1, att_dst1)


def _epi0_kernel(acc_ref, dnm_ref, bias_ref, g_ref, b_ref, o_ref):
    d = dnm_ref[0] + dnm_ref[1]
    vs = []
    s1 = None
    s2 = None
    for h in range(8):
        v = acc_ref[h] / (d[:, h:h + 1] + 1e-16) + bias_ref[h]
        vs.append(v)
        t1 = jnp.sum(v, axis=1, keepdims=True)
        t2 = jnp.sum(v * v, axis=1, keepdims=True)
        s1 = t1 if s1 is None else s1 + t1
        s2 = t2 if s2 is None else s2 + t2
    mu = s1 * (1.0 / 1024.0)
    var = s2 * (1.0 / 1024.0) - mu * mu
    rstd = lax.rsqrt(var + 1e-5)
    for h in range(8):
        yn = (vs[h] - mu) * rstd * g_ref[h] + b_ref[h]
        o_ref[h] = jnp.maximum(yn, 0.0)


def _epi0(accT, dnm, bias0, gamma0, beta0):
    R = 400
    b = bias0.reshape(8, 1, 128)
    g = gamma0.reshape(8, 1, 128)
    bb = beta0.reshape(8, 1, 128)
    return pl.pallas_call(
        _epi0_kernel,
        grid=(N // R,),
        in_specs=[
            pl.BlockSpec((8, R, 128), lambda i: (0, i, 0)),
            pl.BlockSpec((2, R, 16), lambda i: (0, i, 0)),
            pl.BlockSpec((8, 1, 128), lambda i: (0, 0, 0)),
            pl.BlockSpec((8, 1, 128), lambda i: (0, 0, 0)),
            pl.BlockSpec((8, 1, 128), lambda i: (0, 0, 0)),
        ],
        out_specs=pl.BlockSpec((8, R, 128), lambda i: (0, i, 0)),
        out_shape=jax.ShapeDtypeStruct((8, N, 128), _f32),
    )(accT, dnm, b, g, bb)


def _epi1_kernel(p_ref, dnm_ref, bias_ref, g_ref, b_ref, o_ref):
    d = dnm_ref[0] + dnm_ref[1]
    s = (p_ref[0] + p_ref[1]) / (d[:, 0:1] + 1e-16) + bias_ref[...]
    mu = jnp.mean(s, axis=1, keepdims=True)
    var = jnp.mean(s * s, axis=1, keepdims=True) - mu * mu
    o_ref[...] = (s - mu) * lax.rsqrt(var + 1e-5) * g_ref[...] + b_ref[...]


def _epi1(p, dnm, bias1, gamma1, beta1):
    R = 400
    return pl.pallas_call(
        _epi1_kernel,
        grid=(N // R,),
        in_specs=[
            pl.BlockSpec((2, R, 128), lambda i: (0, i, 0)),
            pl.BlockSpec((2, R, 16), lambda i: (0, i, 0)),
            pl.BlockSpec((1, 128), lambda i: (0, 0)),
            pl.BlockSpec((1, 128), lambda i: (0, 0)),
            pl.BlockSpec((1, 128), lambda i: (0, 0)),
        ],
        out_specs=pl.BlockSpec((R, 128), lambda i: (i, 0)),
        out_shape=jax.ShapeDtypeStruct((N, 128), _f32),
    )(p, dnm, bias1.reshape(1, 128), gamma1.reshape(1, 128),
      beta1.reshape(1, 128))


def _zero_vmem(zb, rows, cols):
    zv = jnp.zeros((LANES,), _f32)

    def body(i, _):
        for j in range(cols // LANES):
            zb[i, pl.ds(j * LANES, LANES)] = zv
        return 0

    lax.fori_loop(0, rows, body, 0)


def _make_pass_a():
    mesh = plsc.VectorSubcoreMesh(core_axis_name="c", subcore_axis_name="s")
    e_per_worker = E // (NC * NS)
    nchunks = e_per_worker // CA

    @functools.partial(
        pl.kernel,
        out_type=[
            jax.ShapeDtypeStruct((E, 16), _f32),
            jax.ShapeDtypeStruct((NC, NP, 16), _f32),
        ],
        mesh=mesh,
        compiler_params=pltpu.CompilerParams(use_tc_tiling_on_sc=False),
        scratch_types=[
            pltpu.VMEM((CA,), _i32),
            pltpu.VMEM((CA,), _i32),
            pltpu.VMEM((CA, 16), _f32),
            pltpu.VMEM((CA, 16), _f32),
            pltpu.VMEM((CA, 16), _f32),
            pltpu.VMEM((ZROWS, 16), _f32),
            pltpu.VMEM_SHARED((NP, 16), _f32),
            pltpu.SemaphoreType.DMA,
        ],
    )
    def pass_a(srce_ref, dste_ref, asrc_ref, adst_ref, exr_ref, dnm_ref,
               src_v, dst_v, ga_v, gb_v, ex_v, zb_v, dsh, sem):
        core = lax.axis_index("c")
        sub = lax.axis_index("s")
        base = core * (E // NC) + sub * e_per_worker

        _zero_vmem(zb_v, ZROWS, 16)
        for z in range(ROWS_PER_SUB // ZROWS):
            pltpu.sync_copy(
                zb_v, dsh.at[pl.ds(sub * ROWS_PER_SUB + z * ZROWS, ZROWS), :])
        plsc.subcore_barrier()

        def chunk(c, _):
            off = base + c * CA
            pltpu.sync_copy(srce_ref.at[pl.ds(off, CA)], src_v)
            pltpu.sync_copy(dste_ref.at[pl.ds(off, CA)], dst_v)
            cp1 = pltpu.async_copy(asrc_ref.at[src_v], ga_v, sem)
            cp2 = pltpu.async_copy(adst_ref.at[dst_v], gb_v, sem)
            cp1.wait()
            cp2.wait()
            for i in range(CA):
                v = ga_v[i, :] + gb_v[i, :]
                v = jnp.where(v >= 0.0, v, 0.2 * v)
                ex_v[i, :] = jnp.exp(v)
            pltpu.sync_copy(ex_v, dsh.at[dst_v], add=True)
            pltpu.sync_copy(ex_v, exr_ref.at[pl.ds(off, CA), :])
            return 0

        lax.fori_loop(0, nchunks, chunk, 0)
        plsc.subcore_barrier()

        pltpu.sync_copy(
            dsh.at[pl.ds(sub * ROWS_PER_SUB, ROWS_PER_SUB), :],
            dnm_ref.at[core, pl.ds(sub * ROWS_PER_SUB, ROWS_PER_SUB), :])

    return pass_a


def _make_pass_b(split_heads):
    mesh = plsc.VectorSubcoreMesh(core_axis_name="c", subcore_axis_name="s")
    if split_heads:
        hpc = 4
        e_per_worker = E // NS
        out_major = 8
    else:
        hpc = 1
        e_per_worker = E // (NC * NS)
        out_major = NC
    nchunks = e_per_worker // CB

    @functools.partial(
        pl.kernel,
        out_type=jax.ShapeDtypeStruct((out_major, NP, 128), _f32),
        mesh=mesh,
        compiler_params=pltpu.CompilerParams(use_tc_tiling_on_sc=False),
        scratch_types=[
            pltpu.VMEM((CB,), _i32),
            pltpu.VMEM((CB,), _i32),
            pltpu.VMEM((CB,), _i32),
            pltpu.VMEM((CB, 16), _f32),
            pltpu.VMEM((CB, 128), _f32),
            pltpu.VMEM((ZROWS, 128), _f32),
            pltpu.VMEM_SHARED((NP, 128), _f32),
            pltpu.SemaphoreType.DMA,
        ],
    )
    def pass_b(h_ref, ext_ref, srce_ref, dste_ref, out_ref,
               src_v, dst_v, gidx_v, ex_v, rows_v, zb_v, ash, sem):
        core = lax.axis_index("c")
        sub = lax.axis_index("s")
        _zero_vmem(zb_v, ZROWS, 128)

        def one_head(hl, _):
            if split_heads:
                h = core * hpc + hl
                ebase = sub * e_per_worker
            else:
                h = hl
                ebase = core * (E // NC) + sub * e_per_worker
            for z in range(ROWS_PER_SUB // ZROWS):
                pltpu.sync_copy(
                    zb_v,
                    ash.at[pl.ds(sub * ROWS_PER_SUB + z * ZROWS, ZROWS), :])
            plsc.subcore_barrier()

            def chunk(c, _):
                off = ebase + c * CB
                pltpu.sync_copy(srce_ref.at[pl.ds(off, CB)], src_v)
                pltpu.sync_copy(dste_ref.at[pl.ds(off, CB)], dst_v)
                pltpu.sync_copy(ext_ref.at[pl.ds(off, CB), :], ex_v)
                hoff = h * N
                for g in range(CB // LANES):
                    gidx_v[pl.ds(g * LANES, LANES)] = (
                        src_v[pl.ds(g * LANES, LANES)] + hoff)
                pltpu.async_copy(h_ref.at[gidx_v], rows_v, sem).wait()

                hvec = jnp.zeros((LANES,), _i32) + h

                def scale(i, _):
                    w = ex_v[i, :][hvec]
                    for q in range(128 // LANES):
                        sl = pl.ds(q * LANES, LANES)
                        rows_v[i, sl] = rows_v[i, sl] * w
                    return 0

                lax.fori_loop(0, CB, scale, 0)
                pltpu.sync_copy(rows_v, ash.at[dst_v], add=True)
                return 0

            lax.fori_loop(0, nchunks, chunk, 0)
            plsc.subcore_barrier()
            dst_major = h if split_heads else core
            pltpu.sync_copy(
                ash.at[pl.ds(sub * ROWS_PER_SUB, ROWS_PER_SUB), :],
                out_ref.at[dst_major,
                           pl.ds(sub * ROWS_PER_SUB, ROWS_PER_SUB), :])
            plsc.subcore_barrier()
            return 0

        lax.fori_loop(0, hpc, one_head, 0)

    return pass_b


_pass_a = _make_pass_a()
_pass_b0 = _make_pass_b(True)
_pass_b1 = _make_pass_b(False)


def kernel(x, edge_index, W0, att_src0, att_dst0, bias0, gamma0, beta0,
           W1, att_src1, att_dst1, bias1, gamma1, beta1):
    hT = _mm0(x, W0)
    asrc0, adst0 = _amm0(hT, att_src0, att_dst0)
    src = edge_index[0]
    dst = edge_index[1]
    exr0, dnm0 = _pass_a(src, dst, asrc0, adst0)
    accT = _pass_b0(hT.reshape(8 * N, 128), exr0, src, dst)
    yT = _epi0(accT, dnm0, bias0, gamma0, beta0)

    h1 = _mm1(yT, W1.reshape(8, 128, 128))
    asrc1, adst1 = _amm1(h1, att_src1, att_dst1)
    exr1, dnm1 = _pass_a(src, dst, asrc1, adst1)
    p = _pass_b1(h1, exr1, src, dst)
    out = _epi1(p, dnm1, bias1, gamma1, beta1)
    return out


# speedup vs baseline: 23.7957x; 2.4340x over previous
import functools

import jax
import jax.numpy as jnp
from jax import lax
from jax.experimental import pallas as pl
from jax.experimental.pallas import tpu as pltpu
from jax.experimental.pallas import tpu_sc as plsc

N = 10000
NP = 10240
E = 320000
NC = 2
NS = 16
LANES = 16
CA = 80
CB = 80
ROWS_PER_SUB = NP // NS
ZROWS = 128

_f32 = jnp.float32
_i32 = jnp.int32


def _mm0_kernel(x_ref, w_ref, o_ref):
    o_ref[0] = jnp.dot(x_ref[...], w_ref[...], preferred_element_type=_f32)


def _mm0(x, W0):
    R = 1000
    return pl.pallas_call(
        _mm0_kernel,
        grid=(8, N // R),
        in_specs=[
            pl.BlockSpec((R, 128), lambda h, i: (i, 0)),
            pl.BlockSpec((128, 128), lambda h, i: (0, h)),
        ],
        out_specs=pl.BlockSpec((1, R, 128), lambda h, i: (h, i, 0)),
        out_shape=jax.ShapeDtypeStruct((8, N, 128), _f32),
    )(x, W0)


def _amm0_kernel(hT_ref, asrc_ref, adst_ref, osrc_ref, odst_ref):
    h = pl.program_id(1)
    onehot = (lax.broadcasted_iota(_i32, (1, 16), 1) == h).astype(_f32)
    rs = jnp.sum(hT_ref[0] * asrc_ref[0], axis=1, keepdims=True)
    rd = jnp.sum(hT_ref[0] * adst_ref[0], axis=1, keepdims=True)

    @pl.when(h == 0)
    def _():
        osrc_ref[...] = jnp.zeros_like(osrc_ref)
        odst_ref[...] = jnp.zeros_like(odst_ref)

    osrc_ref[...] += rs * onehot
    odst_ref[...] += rd * onehot


def _amm0(hT, att_src0, att_dst0):
    R = 1000
    return pl.pallas_call(
        _amm0_kernel,
        grid=(N // R, 8),
        in_specs=[
            pl.BlockSpec((1, R, 128), lambda i, h: (h, i, 0)),
            pl.BlockSpec((1, 1, 128), lambda i, h: (h, 0, 0)),
            pl.BlockSpec((1, 1, 128), lambda i, h: (h, 0, 0)),
        ],
        out_specs=[
            pl.BlockSpec((R, 16), lambda i, h: (i, 0)),
            pl.BlockSpec((R, 16), lambda i, h: (i, 0)),
        ],
        out_shape=[
            jax.ShapeDtypeStruct((NP, 16), _f32),
            jax.ShapeDtypeStruct((NP, 16), _f32),
        ],
    )(hT, att_src0.reshape(8, 1, 128), att_dst0.reshape(8, 1, 128))


def _mm1_kernel(y_ref, w_ref, o_ref):
    h = pl.program_id(1)
    acc = jnp.dot(y_ref[0], w_ref[0], preferred_element_type=_f32)

    @pl.when(h == 0)
    def _():
        o_ref[...] = acc

    @pl.when(h != 0)
    def _():
        o_ref[...] += acc


def _mm1(yT, W1r):
    R = 1000
    return pl.pallas_call(
        _mm1_kernel,
        grid=(N // R, 8),
        in_specs=[
            pl.BlockSpec((1, R, 128), lambda i, h: (h, i, 0)),
            pl.BlockSpec((1, 128, 128), lambda i, h: (h, 0, 0)),
        ],
        out_specs=pl.BlockSpec((R, 128), lambda i, h: (i, 0)),
        out_shape=jax.ShapeDtypeStruct((N, 128), _f32),
    )(yT, W1r)


def _amm1_kernel(h1_ref, asrc_ref, adst_ref, osrc_ref, odst_ref):
    onehot0 = (lax.broadcasted_iota(_i32, (1, 16), 1) == 0).astype(_f32)
    rs = jnp.sum(h1_ref[...] * asrc_ref[...], axis=1, keepdims=True)
    rd = jnp.sum(h1_ref[...] * adst_ref[...], axis=1, keepdims=True)
    osrc_ref[...] = rs * onehot0
    odst_ref[...] = rd * onehot0


def _amm1(h1, att_src1, att_dst1):
    R = 1000
    return pl.pallas_call(
        _amm1_kernel,
        grid=(N // R,),
        in_specs=[
            pl.BlockSpec((R, 128), lambda i: (i, 0)),
            pl.BlockSpec((1, 128), lambda i: (0, 0)),
            pl.BlockSpec((1, 128), lambda i: (0, 0)),
        ],
        out_specs=[
            pl.BlockSpec((R, 16), lambda i: (i, 0)),
            pl.BlockSpec((R, 16), lambda i: (i, 0)),
        ],
        out_shape=[
            jax.ShapeDtypeStruct((NP, 16), _f32),
            jax.ShapeDtypeStruct((NP, 16), _f32),
        ],
    )(h1, att_src1, att_dst1)


def _epi0_kernel(acc_ref, dnm_ref, bias_ref, g_ref, b_ref, o_ref):
    d = dnm_ref[0] + dnm_ref[1]
    vs = []
    s1 = None
    s2 = None
    for h in range(8):
        v = acc_ref[h] / (d[:, h:h + 1] + 1e-16) + bias_ref[h]
        vs.append(v)
        t1 = jnp.sum(v, axis=1, keepdims=True)
        t2 = jnp.sum(v * v, axis=1, keepdims=True)
        s1 = t1 if s1 is None else s1 + t1
        s2 = t2 if s2 is None else s2 + t2
    mu = s1 * (1.0 / 1024.0)
    var = s2 * (1.0 / 1024.0) - mu * mu
    rstd = lax.rsqrt(var + 1e-5)
    for h in range(8):
        yn = (vs[h] - mu) * rstd * g_ref[h] + b_ref[h]
        o_ref[h] = jnp.maximum(yn, 0.0)


def _epi0(accT, dnm, bias0, gamma0, beta0):
    R = 400
    b = bias0.reshape(8, 1, 128)
    g = gamma0.reshape(8, 1, 128)
    bb = beta0.reshape(8, 1, 128)
    return pl.pallas_call(
        _epi0_kernel,
        grid=(N // R,),
        in_specs=[
            pl.BlockSpec((8, R, 128), lambda i: (0, i, 0)),
            pl.BlockSpec((2, R, 16), lambda i: (0, i, 0)),
            pl.BlockSpec((8, 1, 128), lambda i: (0, 0, 0)),
            pl.BlockSpec((8, 1, 128), lambda i: (0, 0, 0)),
            pl.BlockSpec((8, 1, 128), lambda i: (0, 0, 0)),
        ],
        out_specs=pl.BlockSpec((8, R, 128), lambda i: (0, i, 0)),
        out_shape=jax.ShapeDtypeStruct((8, N, 128), _f32),
    )(accT, dnm, b, g, bb)


def _epi1_kernel(p_ref, dnm_ref, bias_ref, g_ref, b_ref, o_ref):
    d = dnm_ref[0] + dnm_ref[1]
    s = (p_ref[0] + p_ref[1]) / (d[:, 0:1] + 1e-16) + bias_ref[...]
    mu = jnp.mean(s, axis=1, keepdims=True)
    var = jnp.mean(s * s, axis=1, keepdims=True) - mu * mu
    o_ref[...] = (s - mu) * lax.rsqrt(var + 1e-5) * g_ref[...] + b_ref[...]


def _epi1(p, dnm, bias1, gamma1, beta1):
    R = 400
    return pl.pallas_call(
        _epi1_kernel,
        grid=(N // R,),
        in_specs=[
            pl.BlockSpec((2, R, 128), lambda i: (0, i, 0)),
            pl.BlockSpec((2, R, 16), lambda i: (0, i, 0)),
            pl.BlockSpec((1, 128), lambda i: (0, 0)),
            pl.BlockSpec((1, 128), lambda i: (0, 0)),
            pl.BlockSpec((1, 128), lambda i: (0, 0)),
        ],
        out_specs=pl.BlockSpec((R, 128), lambda i: (i, 0)),
        out_shape=jax.ShapeDtypeStruct((N, 128), _f32),
    )(p, dnm, bias1.reshape(1, 128), gamma1.reshape(1, 128),
      beta1.reshape(1, 128))


def _zero_vmem(zb, rows, cols):
    zv = jnp.zeros((LANES,), _f32)

    def body(i, _):
        for j in range(cols // LANES):
            zb[i, pl.ds(j * LANES, LANES)] = zv
        return 0

    lax.fori_loop(0, rows, body, 0)


def _make_pass_a():
    mesh = plsc.VectorSubcoreMesh(core_axis_name="c", subcore_axis_name="s")
    e_per_worker = E // (NC * NS)
    nchunks = e_per_worker // CA

    @functools.partial(
        pl.kernel,
        out_type=[
            jax.ShapeDtypeStruct((E, 16), _f32),
            jax.ShapeDtypeStruct((NC, NP, 16), _f32),
        ],
        mesh=mesh,
        compiler_params=pltpu.CompilerParams(use_tc_tiling_on_sc=False),
        scratch_types=[
            pltpu.VMEM((CA,), _i32),
            pltpu.VMEM((CA,), _i32),
            pltpu.VMEM((CA, 16), _f32),
            pltpu.VMEM((CA, 16), _f32),
            pltpu.VMEM((CA, 16), _f32),
            pltpu.VMEM((ZROWS, 16), _f32),
            pltpu.VMEM_SHARED((NP, 16), _f32),
            pltpu.SemaphoreType.DMA,
        ],
    )
    def pass_a(srce_ref, dste_ref, asrc_ref, adst_ref, exr_ref, dnm_ref,
               src_v, dst_v, ga_v, gb_v, ex_v, zb_v, dsh, sem):
        core = lax.axis_index("c")
        sub = lax.axis_index("s")
        base = core * (E // NC) + sub * e_per_worker

        _zero_vmem(zb_v, ZROWS, 16)
        for z in range(ROWS_PER_SUB // ZROWS):
            pltpu.sync_copy(
                zb_v, dsh.at[pl.ds(sub * ROWS_PER_SUB + z * ZROWS, ZROWS), :])
        plsc.subcore_barrier()

        def chunk(c, _):
            off = base + c * CA
            pltpu.sync_copy(srce_ref.at[pl.ds(off, CA)], src_v)
            pltpu.sync_copy(dste_ref.at[pl.ds(off, CA)], dst_v)
            cp1 = pltpu.async_copy(asrc_ref.at[src_v], ga_v, sem)
            cp2 = pltpu.async_copy(adst_ref.at[dst_v], gb_v, sem)
            cp1.wait()
            cp2.wait()
            for i in range(CA):
                v = ga_v[i, :] + gb_v[i, :]
                v = jnp.where(v >= 0.0, v, 0.2 * v)
                ex_v[i, :] = jnp.exp(v)
            pltpu.sync_copy(ex_v, dsh.at[dst_v], add=True)
            pltpu.sync_copy(ex_v, exr_ref.at[pl.ds(off, CA), :])
            return 0

        lax.fori_loop(0, nchunks, chunk, 0)
        plsc.subcore_barrier()

        pltpu.sync_copy(
            dsh.at[pl.ds(sub * ROWS_PER_SUB, ROWS_PER_SUB), :],
            dnm_ref.at[core, pl.ds(sub * ROWS_PER_SUB, ROWS_PER_SUB), :])

    return pass_a


def _make_pass_b(split_heads):
    mesh = plsc.VectorSubcoreMesh(core_axis_name="c", subcore_axis_name="s")
    if split_heads:
        hpc = 4
        e_per_worker = E // NS
        out_major = 8
    else:
        hpc = 1
        e_per_worker = E // (NC * NS)
        out_major = NC
    nchunks = e_per_worker // CB
    NBUF = 4
    nstages = -(-(nchunks + 2) // NBUF) * NBUF
    nsteps = nstages // NBUF

    gissued = [0] * NBUF
    gwaited = [0] * NBUF
    for c in (0, 1):
        gissued[c % NBUF] += 1
    for m in range(nstages):
        gissued[(m + 2) % NBUF] += 1
        if m < nchunks:
            gwaited[m % NBUF] += 1

    scratch = []
    for _ in range(NBUF):
        scratch += [
            pltpu.VMEM((CB,), _i32),
            pltpu.VMEM((CB,), _i32),
            pltpu.VMEM((CB,), _i32),
            pltpu.VMEM((CB, 16), _f32),
            pltpu.VMEM((CB, 128), _f32),
        ]
    scratch += [pltpu.VMEM_SHARED((NP, 128), _f32)]
    scratch += [pltpu.SemaphoreType.DMA] * (3 * NBUF)

    @functools.partial(
        pl.kernel,
        out_type=jax.ShapeDtypeStruct((out_major, NP, 128), _f32),
        mesh=mesh,
        compiler_params=pltpu.CompilerParams(use_tc_tiling_on_sc=False),
        scratch_types=scratch,
    )
    def pass_b(h_ref, ext_ref, srce_ref, dste_ref, zeros_ref, out_ref, *scr):
        src_b = [scr[5 * b + 0] for b in range(NBUF)]
        dst_b = [scr[5 * b + 1] for b in range(NBUF)]
        gidx_b = [scr[5 * b + 2] for b in range(NBUF)]
        ex_b = [scr[5 * b + 3] for b in range(NBUF)]
        rows_b = [scr[5 * b + 4] for b in range(NBUF)]
        ash = scr[5 * NBUF]
        gsem = scr[5 * NBUF + 1:5 * NBUF + 1 + NBUF]
        ssem = scr[5 * NBUF + 1 + NBUF:5 * NBUF + 1 + 2 * NBUF]
        isem = scr[5 * NBUF + 1 + 2 * NBUF:5 * NBUF + 1 + 3 * NBUF]

        core = lax.axis_index("c")
        sub = lax.axis_index("s")

        def idx_issue(b, cc, sync):
            off = ebase[0] + cc * CB
            if sync:
                pltpu.sync_copy(srce_ref.at[pl.ds(off, CB)], src_b[b])
                pltpu.sync_copy(dste_ref.at[pl.ds(off, CB)], dst_b[b])
                pltpu.sync_copy(ext_ref.at[pl.ds(off, CB), :], ex_b[b])
            else:
                pltpu.async_copy(srce_ref.at[pl.ds(off, CB)], src_b[b],
                                 isem[b])
                pltpu.async_copy(dste_ref.at[pl.ds(off, CB)], dst_b[b],
                                 isem[b])
                pltpu.async_copy(ext_ref.at[pl.ds(off, CB), :], ex_b[b],
                                 isem[b])

        def idx_wait(b):
            pltpu.make_async_copy(srce_ref.at[pl.ds(0, CB)], src_b[b],
                                  isem[b]).wait()
            pltpu.make_async_copy(dste_ref.at[pl.ds(0, CB)], dst_b[b],
                                  isem[b]).wait()
            pltpu.make_async_copy(ext_ref.at[pl.ds(0, CB), :], ex_b[b],
                                  isem[b]).wait()

        def gather_issue(b, hoff):
            for g in range(CB // LANES):
                sl = pl.ds(g * LANES, LANES)
                gidx_b[b][sl] = src_b[b][sl] + hoff
            pltpu.async_copy(h_ref.at[gidx_b[b]], rows_b[b], gsem[b])

        ebase = [jnp.int32(0)]

        def one_head(hl, _):
            if split_heads:
                h = core * hpc + hl
                ebase[0] = sub * e_per_worker
            else:
                h = hl
                ebase[0] = core * (E // NC) + sub * e_per_worker
            hvec = jnp.zeros((LANES,), _i32) + h
            hoff = h * N

            for z in range(ROWS_PER_SUB // ZROWS):
                pltpu.sync_copy(
                    zeros_ref,
                    ash.at[pl.ds(sub * ROWS_PER_SUB + z * ZROWS, ZROWS), :])
            plsc.subcore_barrier()

            idx_issue(0, jnp.int32(0), True)
            gather_issue(0, hoff)
            idx_issue(1, jnp.int32(1), True)
            gather_issue(1, hoff)

            def tloop(t, _):
                for j in range(NBUF):
                    m = t * NBUF + j
                    b2 = (j + 2) % NBUF
                    bm = j

                    @pl.when(jnp.logical_and(m >= 2, m - 2 < nchunks))
                    def _():
                        pltpu.make_async_copy(
                            rows_b[b2], ash.at[dst_b[b2]], ssem[b2]).wait()

                    idx_issue(b2, jnp.minimum(m + 2, nchunks - 1), False)

                    @pl.when(m < nchunks)
                    def _():
                        pltpu.make_async_copy(
                            h_ref.at[gidx_b[bm]], rows_b[bm],
                            gsem[bm]).wait()
                        rv = rows_b[bm]
                        ev = ex_b[bm]

                        def scale(g, _):
                            for jj in range(4):
                                i = g * 4 + jj
                                w = ev[i, :][hvec]
                                for q in range(128 // LANES):
                                    sl = pl.ds(q * LANES, LANES)
                                    rv[i, sl] = rv[i, sl] * w
                            return 0

                        lax.fori_loop(0, CB // 4, scale, 0)
                        pltpu.async_copy(rv, ash.at[dst_b[bm]], ssem[bm],
                                         add=True)

                    idx_wait(b2)
                    gather_issue(b2, hoff)
                return 0

            lax.fori_loop(0, nsteps, tloop, 0)

            for b in range(NBUF):
                for _ in range(gissued[b] - gwaited[b]):
                    pltpu.make_async_copy(
                        h_ref.at[gidx_b[b]], rows_b[b], gsem[b]).wait()
            plsc.subcore_barrier()
            dst_major = h if split_heads else core
            pltpu.sync_copy(
                ash.at[pl.ds(sub * ROWS_PER_SUB, ROWS_PER_SUB), :],
                out_ref.at[dst_major,
                           pl.ds(sub * ROWS_PER_SUB, ROWS_PER_SUB), :])
            plsc.subcore_barrier()
            return 0

        lax.fori_loop(0, hpc, one_head, 0)

    return pass_b


_pass_a = _make_pass_a()
_pass_b0 = _make_pass_b(True)
_pass_b1 = _make_pass_b(False)


def kernel(x, edge_index, W0, att_src0, att_dst0, bias0, gamma0, beta0,
           W1, att_src1, att_dst1, bias1, gamma1, beta1):
    hT = _mm0(x, W0)
    asrc0, adst0 = _amm0(hT, att_src0, att_dst0)
    src = edge_index[0]
    dst = edge_index[1]
    exr0, dnm0 = _pass_a(src, dst, asrc0, adst0)
    zeros = jnp.zeros((ZROWS, 128), _f32)
    accT = _pass_b0(hT.reshape(8 * N, 128), exr0, src, dst, zeros)
    yT = _epi0(accT, dnm0, bias0, gamma0, beta0)

    h1 = _mm1(yT, W1.reshape(8, 128, 128))
    asrc1, adst1 = _amm1(h1, att_src1, att_dst1)
    exr1, dnm1 = _pass_a(src, dst, asrc1, adst1)
    p = _pass_b1(h1, exr1, src, dst, zeros)
    out = _epi1(p, dnm1, bias1, gamma1, beta1)
    return out


# speedup vs baseline: 28.1544x; 1.1832x over previous
import functools

import jax
import jax.numpy as jnp
from jax import lax
from jax.experimental import pallas as pl
from jax.experimental.pallas import tpu as pltpu
from jax.experimental.pallas import tpu_sc as plsc

N = 10000
NP = 10240
E = 320000
NC = 2
NS = 16
LANES = 16
CA = 80
CB = 80
ROWS_PER_SUB = NP // NS
ZROWS = 128

_f32 = jnp.float32
_i32 = jnp.int32


def _mm0_kernel(x_ref, w_ref, o_ref):
    o_ref[0] = jnp.dot(x_ref[...], w_ref[...], preferred_element_type=_f32)


def _mm0(x, W0):
    R = 1000
    return pl.pallas_call(
        _mm0_kernel,
        grid=(8, N // R),
        in_specs=[
            pl.BlockSpec((R, 128), lambda h, i: (i, 0)),
            pl.BlockSpec((128, 128), lambda h, i: (0, h)),
        ],
        out_specs=pl.BlockSpec((1, R, 128), lambda h, i: (h, i, 0)),
        out_shape=jax.ShapeDtypeStruct((8, N, 128), _f32),
    )(x, W0)


def _amm0_kernel(hT_ref, asrc_ref, adst_ref, osrc_ref, odst_ref):
    h = pl.program_id(1)
    onehot = (lax.broadcasted_iota(_i32, (1, 16), 1) == h).astype(_f32)
    rs = jnp.sum(hT_ref[0] * asrc_ref[0], axis=1, keepdims=True)
    rd = jnp.sum(hT_ref[0] * adst_ref[0], axis=1, keepdims=True)

    @pl.when(h == 0)
    def _():
        osrc_ref[...] = jnp.zeros_like(osrc_ref)
        odst_ref[...] = jnp.zeros_like(odst_ref)

    osrc_ref[...] += rs * onehot
    odst_ref[...] += rd * onehot


def _amm0(hT, att_src0, att_dst0):
    R = 1000
    return pl.pallas_call(
        _amm0_kernel,
        grid=(N // R, 8),
        in_specs=[
            pl.BlockSpec((1, R, 128), lambda i, h: (h, i, 0)),
            pl.BlockSpec((1, 1, 128), lambda i, h: (h, 0, 0)),
            pl.BlockSpec((1, 1, 128), lambda i, h: (h, 0, 0)),
        ],
        out_specs=[
            pl.BlockSpec((R, 16), lambda i, h: (i, 0)),
            pl.BlockSpec((R, 16), lambda i, h: (i, 0)),
        ],
        out_shape=[
            jax.ShapeDtypeStruct((NP, 16), _f32),
            jax.ShapeDtypeStruct((NP, 16), _f32),
        ],
    )(hT, att_src0.reshape(8, 1, 128), att_dst0.reshape(8, 1, 128))


def _mm1_kernel(y_ref, w_ref, o_ref):
    h = pl.program_id(1)
    acc = jnp.dot(y_ref[0], w_ref[0], preferred_element_type=_f32)

    @pl.when(h == 0)
    def _():
        o_ref[...] = acc

    @pl.when(h != 0)
    def _():
        o_ref[...] += acc


def _mm1(yT, W1r):
    R = 1000
    return pl.pallas_call(
        _mm1_kernel,
        grid=(N // R, 8),
        in_specs=[
            pl.BlockSpec((1, R, 128), lambda i, h: (h, i, 0)),
            pl.BlockSpec((1, 128, 128), lambda i, h: (h, 0, 0)),
        ],
        out_specs=pl.BlockSpec((R, 128), lambda i, h: (i, 0)),
        out_shape=jax.ShapeDtypeStruct((N, 128), _f32),
    )(yT, W1r)


def _amm1_kernel(h1_ref, asrc_ref, adst_ref, osrc_ref, odst_ref):
    onehot0 = (lax.broadcasted_iota(_i32, (1, 16), 1) == 0).astype(_f32)
    rs = jnp.sum(h1_ref[...] * asrc_ref[...], axis=1, keepdims=True)
    rd = jnp.sum(h1_ref[...] * adst_ref[...], axis=1, keepdims=True)
    osrc_ref[...] = rs * onehot0
    odst_ref[...] = rd * onehot0


def _amm1(h1, att_src1, att_dst1):
    R = 1000
    return pl.pallas_call(
        _amm1_kernel,
        grid=(N // R,),
        in_specs=[
            pl.BlockSpec((R, 128), lambda i: (i, 0)),
            pl.BlockSpec((1, 128), lambda i: (0, 0)),
            pl.BlockSpec((1, 128), lambda i: (0, 0)),
        ],
        out_specs=[
            pl.BlockSpec((R, 16), lambda i: (i, 0)),
            pl.BlockSpec((R, 16), lambda i: (i, 0)),
        ],
        out_shape=[
            jax.ShapeDtypeStruct((NP, 16), _f32),
            jax.ShapeDtypeStruct((NP, 16), _f32),
        ],
    )(h1, att_src1, att_dst1)


def _epi0_kernel(acc_ref, dnm_ref, bias_ref, g_ref, b_ref, o_ref):
    d = dnm_ref[0] + dnm_ref[1]
    vs = []
    s1 = None
    s2 = None
    for h in range(8):
        v = acc_ref[h] / (d[:, h:h + 1] + 1e-16) + bias_ref[h]
        vs.append(v)
        t1 = jnp.sum(v, axis=1, keepdims=True)
        t2 = jnp.sum(v * v, axis=1, keepdims=True)
        s1 = t1 if s1 is None else s1 + t1
        s2 = t2 if s2 is None else s2 + t2
    mu = s1 * (1.0 / 1024.0)
    var = s2 * (1.0 / 1024.0) - mu * mu
    rstd = lax.rsqrt(var + 1e-5)
    for h in range(8):
        yn = (vs[h] - mu) * rstd * g_ref[h] + b_ref[h]
        o_ref[h] = jnp.maximum(yn, 0.0)


def _epi0(accT, dnm, bias0, gamma0, beta0):
    R = 400
    b = bias0.reshape(8, 1, 128)
    g = gamma0.reshape(8, 1, 128)
    bb = beta0.reshape(8, 1, 128)
    return pl.pallas_call(
        _epi0_kernel,
        grid=(N // R,),
        in_specs=[
            pl.BlockSpec((8, R, 128), lambda i: (0, i, 0)),
            pl.BlockSpec((2, R, 16), lambda i: (0, i, 0)),
            pl.BlockSpec((8, 1, 128), lambda i: (0, 0, 0)),
            pl.BlockSpec((8, 1, 128), lambda i: (0, 0, 0)),
            pl.BlockSpec((8, 1, 128), lambda i: (0, 0, 0)),
        ],
        out_specs=pl.BlockSpec((8, R, 128), lambda i: (0, i, 0)),
        out_shape=jax.ShapeDtypeStruct((8, N, 128), _f32),
    )(accT, dnm, b, g, bb)


def _epi1_kernel(p_ref, dnm_ref, bias_ref, g_ref, b_ref, o_ref):
    d = dnm_ref[0] + dnm_ref[1]
    s = (p_ref[0] + p_ref[1]) / (d[:, 0:1] + 1e-16) + bias_ref[...]
    mu = jnp.mean(s, axis=1, keepdims=True)
    var = jnp.mean(s * s, axis=1, keepdims=True) - mu * mu
    o_ref[...] = (s - mu) * lax.rsqrt(var + 1e-5) * g_ref[...] + b_ref[...]


def _epi1(p, dnm, bias1, gamma1, beta1):
    R = 400
    return pl.pallas_call(
        _epi1_kernel,
        grid=(N // R,),
        in_specs=[
            pl.BlockSpec((2, R, 128), lambda i: (0, i, 0)),
            pl.BlockSpec((2, R, 16), lambda i: (0, i, 0)),
            pl.BlockSpec((1, 128), lambda i: (0, 0)),
            pl.BlockSpec((1, 128), lambda i: (0, 0)),
            pl.BlockSpec((1, 128), lambda i: (0, 0)),
        ],
        out_specs=pl.BlockSpec((R, 128), lambda i: (i, 0)),
        out_shape=jax.ShapeDtypeStruct((N, 128), _f32),
    )(p, dnm, bias1.reshape(1, 128), gamma1.reshape(1, 128),
      beta1.reshape(1, 128))


def _zero_vmem(zb, rows, cols):
    zv = jnp.zeros((LANES,), _f32)

    def body(i, _):
        for j in range(cols // LANES):
            zb[i, pl.ds(j * LANES, LANES)] = zv
        return 0

    lax.fori_loop(0, rows, body, 0)


def _make_pass_a():
    mesh = plsc.VectorSubcoreMesh(core_axis_name="c", subcore_axis_name="s")
    e_per_worker = E // (NC * NS)
    nchunks = e_per_worker // CA
    NBUF = 4
    nstages = -(-(nchunks + 2) // NBUF) * NBUF
    nsteps = nstages // NBUF

    gissued = [0] * NBUF
    gwaited = [0] * NBUF
    for c in (0, 1):
        gissued[c % NBUF] += 1
    for m in range(nstages):
        gissued[(m + 2) % NBUF] += 1
        if m < nchunks:
            gwaited[m % NBUF] += 1

    scratch = []
    for _ in range(NBUF):
        scratch += [
            pltpu.VMEM((CA,), _i32),
            pltpu.VMEM((CA,), _i32),
            pltpu.VMEM((CA, 16), _f32),
            pltpu.VMEM((CA, 16), _f32),
            pltpu.VMEM((CA, 16), _f32),
        ]
    scratch += [pltpu.VMEM_SHARED((NP, 16), _f32)]
    scratch += [pltpu.SemaphoreType.DMA] * (4 * NBUF)

    @functools.partial(
        pl.kernel,
        out_type=[
            jax.ShapeDtypeStruct((E, 16), _f32),
            jax.ShapeDtypeStruct((NC, NP, 16), _f32),
        ],
        mesh=mesh,
        compiler_params=pltpu.CompilerParams(use_tc_tiling_on_sc=False),
        scratch_types=scratch,
    )
    def pass_a(srce_ref, dste_ref, asrc_ref, adst_ref, zeros_ref,
               exr_ref, dnm_ref, *scr):
        src_b = [scr[5 * b + 0] for b in range(NBUF)]
        dst_b = [scr[5 * b + 1] for b in range(NBUF)]
        ga_b = [scr[5 * b + 2] for b in range(NBUF)]
        gb_b = [scr[5 * b + 3] for b in range(NBUF)]
        ex_b = [scr[5 * b + 4] for b in range(NBUF)]
        dsh = scr[5 * NBUF]
        gsem = scr[5 * NBUF + 1:5 * NBUF + 1 + NBUF]
        ssem = scr[5 * NBUF + 1 + NBUF:5 * NBUF + 1 + 2 * NBUF]
        wsem = scr[5 * NBUF + 1 + 2 * NBUF:5 * NBUF + 1 + 3 * NBUF]
        isem = scr[5 * NBUF + 1 + 3 * NBUF:5 * NBUF + 1 + 4 * NBUF]

        core = lax.axis_index("c")
        sub = lax.axis_index("s")
        base = core * (E // NC) + sub * e_per_worker

        for z in range(ROWS_PER_SUB // ZROWS):
            pltpu.sync_copy(
                zeros_ref,
                dsh.at[pl.ds(sub * ROWS_PER_SUB + z * ZROWS, ZROWS), :])
        plsc.subcore_barrier()

        def idx_issue(b, cc, sync):
            off = base + cc * CA
            if sync:
                pltpu.sync_copy(srce_ref.at[pl.ds(off, CA)], src_b[b])
                pltpu.sync_copy(dste_ref.at[pl.ds(off, CA)], dst_b[b])
            else:
                pltpu.async_copy(srce_ref.at[pl.ds(off, CA)], src_b[b],
                                 isem[b])
                pltpu.async_copy(dste_ref.at[pl.ds(off, CA)], dst_b[b],
                                 isem[b])

        def idx_wait(b):
            pltpu.make_async_copy(srce_ref.at[pl.ds(0, CA)], src_b[b],
                                  isem[b]).wait()
            pltpu.make_async_copy(dste_ref.at[pl.ds(0, CA)], dst_b[b],
                                  isem[b]).wait()

        def gather_issue(b):
            pltpu.async_copy(asrc_ref.at[src_b[b]], ga_b[b], gsem[b])
            pltpu.async_copy(adst_ref.at[dst_b[b]], gb_b[b], gsem[b])

        def gather_wait(b):
            pltpu.make_async_copy(asrc_ref.at[src_b[b]], ga_b[b],
                                  gsem[b]).wait()
            pltpu.make_async_copy(adst_ref.at[dst_b[b]], gb_b[b],
                                  gsem[b]).wait()

        idx_issue(0, jnp.int32(0), True)
        gather_issue(0)
        idx_issue(1, jnp.int32(1), True)
        gather_issue(1)

        def tloop(t, _):
            for j in range(NBUF):
                m = t * NBUF + j
                b2 = (j + 2) % NBUF
                bm = j

                @pl.when(jnp.logical_and(m >= 2, m - 2 < nchunks))
                def _():
                    pltpu.make_async_copy(
                        ex_b[b2], dsh.at[dst_b[b2]], ssem[b2]).wait()
                    pltpu.make_async_copy(
                        ex_b[b2], exr_ref.at[pl.ds(0, CA), :],
                        wsem[b2]).wait()

                idx_issue(b2, jnp.minimum(m + 2, nchunks - 1), False)

                @pl.when(m < nchunks)
                def _():
                    gather_wait(bm)
                    ga = ga_b[bm]
                    gb = gb_b[bm]
                    ex = ex_b[bm]

                    def comp(g, _):
                        for jj in range(4):
                            i = g * 4 + jj
                            v = ga[i, :] + gb[i, :]
                            v = jnp.where(v >= 0.0, v, 0.2 * v)
                            ex[i, :] = jnp.exp(v)
                        return 0

                    lax.fori_loop(0, CA // 4, comp, 0)
                    pltpu.async_copy(ex, dsh.at[dst_b[bm]], ssem[bm],
                                     add=True)
                    off = base + m * CA
                    pltpu.async_copy(ex, exr_ref.at[pl.ds(off, CA), :],
                                     wsem[bm])

                idx_wait(b2)
                gather_issue(b2)
            return 0

        lax.fori_loop(0, nsteps, tloop, 0)

        for b in range(NBUF):
            for _ in range(gissued[b] - gwaited[b]):
                gather_wait(b)
        plsc.subcore_barrier()

        pltpu.sync_copy(
            dsh.at[pl.ds(sub * ROWS_PER_SUB, ROWS_PER_SUB), :],
            dnm_ref.at[core, pl.ds(sub * ROWS_PER_SUB, ROWS_PER_SUB), :])

    return pass_a


def _make_pass_b(split_heads):
    mesh = plsc.VectorSubcoreMesh(core_axis_name="c", subcore_axis_name="s")
    if split_heads:
        hpc = 4
        e_per_worker = E // NS
        out_major = 8
    else:
        hpc = 1
        e_per_worker = E // (NC * NS)
        out_major = NC
    nchunks = e_per_worker // CB
    NBUF = 4
    nstages = -(-(nchunks + 2) // NBUF) * NBUF
    nsteps = nstages // NBUF

    gissued = [0] * NBUF
    gwaited = [0] * NBUF
    for c in (0, 1):
        gissued[c % NBUF] += 1
    for m in range(nstages):
        gissued[(m + 2) % NBUF] += 1
        if m < nchunks:
            gwaited[m % NBUF] += 1

    scratch = []
    for _ in range(NBUF):
        scratch += [
            pltpu.VMEM((CB,), _i32),
            pltpu.VMEM((CB,), _i32),
            pltpu.VMEM((CB,), _i32),
            pltpu.VMEM((CB, 16), _f32),
            pltpu.VMEM((CB, 128), _f32),
        ]
    scratch += [pltpu.VMEM_SHARED((NP, 128), _f32)]
    scratch += [pltpu.SemaphoreType.DMA] * (3 * NBUF)

    @functools.partial(
        pl.kernel,
        out_type=jax.ShapeDtypeStruct((out_major, NP, 128), _f32),
        mesh=mesh,
        compiler_params=pltpu.CompilerParams(use_tc_tiling_on_sc=False),
        scratch_types=scratch,
    )
    def pass_b(h_ref, ext_ref, srce_ref, dste_ref, zeros_ref, out_ref, *scr):
        src_b = [scr[5 * b + 0] for b in range(NBUF)]
        dst_b = [scr[5 * b + 1] for b in range(NBUF)]
        gidx_b = [scr[5 * b + 2] for b in range(NBUF)]
        ex_b = [scr[5 * b + 3] for b in range(NBUF)]
        rows_b = [scr[5 * b + 4] for b in range(NBUF)]
        ash = scr[5 * NBUF]
        gsem = scr[5 * NBUF + 1:5 * NBUF + 1 + NBUF]
        ssem = scr[5 * NBUF + 1 + NBUF:5 * NBUF + 1 + 2 * NBUF]
        isem = scr[5 * NBUF + 1 + 2 * NBUF:5 * NBUF + 1 + 3 * NBUF]

        core = lax.axis_index("c")
        sub = lax.axis_index("s")

        def idx_issue(b, cc, sync):
            off = ebase[0] + cc * CB
            if sync:
                pltpu.sync_copy(srce_ref.at[pl.ds(off, CB)], src_b[b])
                pltpu.sync_copy(dste_ref.at[pl.ds(off, CB)], dst_b[b])
                pltpu.sync_copy(ext_ref.at[pl.ds(off, CB), :], ex_b[b])
            else:
                pltpu.async_copy(srce_ref.at[pl.ds(off, CB)], src_b[b],
                                 isem[b])
                pltpu.async_copy(dste_ref.at[pl.ds(off, CB)], dst_b[b],
                                 isem[b])
                pltpu.async_copy(ext_ref.at[pl.ds(off, CB), :], ex_b[b],
                                 isem[b])

        def idx_wait(b):
            pltpu.make_async_copy(srce_ref.at[pl.ds(0, CB)], src_b[b],
                                  isem[b]).wait()
            pltpu.make_async_copy(dste_ref.at[pl.ds(0, CB)], dst_b[b],
                                  isem[b]).wait()
            pltpu.make_async_copy(ext_ref.at[pl.ds(0, CB), :], ex_b[b],
                                  isem[b]).wait()

        def gather_issue(b, hoff):
            for g in range(CB // LANES):
                sl = pl.ds(g * LANES, LANES)
                gidx_b[b][sl] = src_b[b][sl] + hoff
            pltpu.async_copy(h_ref.at[gidx_b[b]], rows_b[b], gsem[b])

        ebase = [jnp.int32(0)]

        def one_head(hl, _):
            if split_heads:
                h = core * hpc + hl
                ebase[0] = sub * e_per_worker
            else:
                h = hl
                ebase[0] = core * (E // NC) + sub * e_per_worker
            hvec = jnp.zeros((LANES,), _i32) + h
            hoff = h * N

            for z in range(ROWS_PER_SUB // ZROWS):
                pltpu.sync_copy(
                    zeros_ref,
                    ash.at[pl.ds(sub * ROWS_PER_SUB + z * ZROWS, ZROWS), :])
            plsc.subcore_barrier()

            idx_issue(0, jnp.int32(0), True)
            gather_issue(0, hoff)
            idx_issue(1, jnp.int32(1), True)
            gather_issue(1, hoff)

            def tloop(t, _):
                for j in range(NBUF):
                    m = t * NBUF + j
                    b2 = (j + 2) % NBUF
                    bm = j

                    @pl.when(jnp.logical_and(m >= 2, m - 2 < nchunks))
                    def _():
                        pltpu.make_async_copy(
                            rows_b[b2], ash.at[dst_b[b2]], ssem[b2]).wait()

                    idx_issue(b2, jnp.minimum(m + 2, nchunks - 1), False)

                    @pl.when(m < nchunks)
                    def _():
                        pltpu.make_async_copy(
                            h_ref.at[gidx_b[bm]], rows_b[bm],
                            gsem[bm]).wait()
                        rv = rows_b[bm]
                        ev = ex_b[bm]

                        def scale(g, _):
                            for jj in range(4):
                                i = g * 4 + jj
                                w = ev[i, :][hvec]
                                for q in range(128 // LANES):
                                    sl = pl.ds(q * LANES, LANES)
                                    rv[i, sl] = rv[i, sl] * w
                            return 0

                        lax.fori_loop(0, CB // 4, scale, 0)
                        pltpu.async_copy(rv, ash.at[dst_b[bm]], ssem[bm],
                                         add=True)

                    idx_wait(b2)
                    gather_issue(b2, hoff)
                return 0

            lax.fori_loop(0, nsteps, tloop, 0)

            for b in range(NBUF):
                for _ in range(gissued[b] - gwaited[b]):
                    pltpu.make_async_copy(
                        h_ref.at[gidx_b[b]], rows_b[b], gsem[b]).wait()
            plsc.subcore_barrier()
            dst_major = h if split_heads else core
            pltpu.sync_copy(
                ash.at[pl.ds(sub * ROWS_PER_SUB, ROWS_PER_SUB), :],
                out_ref.at[dst_major,
                           pl.ds(sub * ROWS_PER_SUB, ROWS_PER_SUB), :])
            plsc.subcore_barrier()
            return 0

        lax.fori_loop(0, hpc, one_head, 0)

    return pass_b


_pass_a = _make_pass_a()
_pass_b0 = _make_pass_b(True)
_pass_b1 = _make_pass_b(False)


def kernel(x, edge_index, W0, att_src0, att_dst0, bias0, gamma0, beta0,
           W1, att_src1, att_dst1, bias1, gamma1, beta1):
    hT = _mm0(x, W0)
    asrc0, adst0 = _amm0(hT, att_src0, att_dst0)
    src = edge_index[0]
    dst = edge_index[1]
    zeros = jnp.zeros((ZROWS, 128), _f32)
    zeros16 = jnp.zeros((ZROWS, 16), _f32)
    exr0, dnm0 = _pass_a(src, dst, asrc0, adst0, zeros16)
    accT = _pass_b0(hT.reshape(8 * N, 128), exr0, src, dst, zeros)
    yT = _epi0(accT, dnm0, bias0, gamma0, beta0)

    h1 = _mm1(yT, W1.reshape(8, 128, 128))
    asrc1, adst1 = _amm1(h1, att_src1, att_dst1)
    exr1, dnm1 = _pass_a(src, dst, asrc1, adst1, zeros16)
    p = _pass_b1(h1, exr1, src, dst, zeros)
    out = _epi1(p, dnm1, bias1, gamma1, beta1)
    return out


# speedup vs baseline: 30.3724x; 1.0788x over previous
import functools

import jax
import jax.numpy as jnp
from jax import lax
from jax.experimental import pallas as pl
from jax.experimental.pallas import tpu as pltpu
from jax.experimental.pallas import tpu_sc as plsc

N = 10000
NP = 10240
E = 320000
NC = 2
NS = 16
LANES = 16
CA = 80
CB = 80
ROWS_PER_SUB = NP // NS
ZROWS = 128

_f32 = jnp.float32
_i32 = jnp.int32


def _mm0_kernel(x_ref, w_ref, asrc_ref, adst_ref, hT_ref, osrc_ref,
                odst_ref):
    h = pl.program_id(1)
    hv = jnp.dot(x_ref[...], w_ref[...], preferred_element_type=_f32)
    hT_ref[0] = hv
    onehot = (lax.broadcasted_iota(_i32, (1, 16), 1) == h).astype(_f32)
    rs = jnp.sum(hv * asrc_ref[0], axis=1, keepdims=True)
    rd = jnp.sum(hv * adst_ref[0], axis=1, keepdims=True)

    @pl.when(h == 0)
    def _():
        osrc_ref[...] = jnp.zeros_like(osrc_ref)
        odst_ref[...] = jnp.zeros_like(odst_ref)

    osrc_ref[...] += rs * onehot
    odst_ref[...] += rd * onehot


def _mm0(x, W0, att_src0, att_dst0):
    R = 1000
    return pl.pallas_call(
        _mm0_kernel,
        grid=(N // R, 8),
        in_specs=[
            pl.BlockSpec((R, 128), lambda i, h: (i, 0)),
            pl.BlockSpec((128, 128), lambda i, h: (0, h)),
            pl.BlockSpec((1, 1, 128), lambda i, h: (h, 0, 0)),
            pl.BlockSpec((1, 1, 128), lambda i, h: (h, 0, 0)),
        ],
        out_specs=[
            pl.BlockSpec((1, R, 128), lambda i, h: (h, i, 0)),
            pl.BlockSpec((R, 16), lambda i, h: (i, 0)),
            pl.BlockSpec((R, 16), lambda i, h: (i, 0)),
        ],
        out_shape=[
            jax.ShapeDtypeStruct((8, N, 128), _f32),
            jax.ShapeDtypeStruct((NP, 16), _f32),
            jax.ShapeDtypeStruct((NP, 16), _f32),
        ],
    )(x, W0, att_src0.reshape(8, 1, 128), att_dst0.reshape(8, 1, 128))


def _mm1_kernel(y_ref, w_ref, o_ref):
    h = pl.program_id(1)
    acc = jnp.dot(y_ref[0], w_ref[0], preferred_element_type=_f32)

    @pl.when(h == 0)
    def _():
        o_ref[...] = acc

    @pl.when(h != 0)
    def _():
        o_ref[...] += acc


def _mm1(yT, W1r):
    R = 1000
    return pl.pallas_call(
        _mm1_kernel,
        grid=(N // R, 8),
        in_specs=[
            pl.BlockSpec((1, R, 128), lambda i, h: (h, i, 0)),
            pl.BlockSpec((1, 128, 128), lambda i, h: (h, 0, 0)),
        ],
        out_specs=pl.BlockSpec((R, 128), lambda i, h: (i, 0)),
        out_shape=jax.ShapeDtypeStruct((N, 128), _f32),
    )(yT, W1r)


def _amm1_kernel(h1_ref, asrc_ref, adst_ref, osrc_ref, odst_ref):
    onehot0 = (lax.broadcasted_iota(_i32, (1, 16), 1) == 0).astype(_f32)
    rs = jnp.sum(h1_ref[...] * asrc_ref[...], axis=1, keepdims=True)
    rd = jnp.sum(h1_ref[...] * adst_ref[...], axis=1, keepdims=True)
    osrc_ref[...] = rs * onehot0
    odst_ref[...] = rd * onehot0


def _amm1(h1, att_src1, att_dst1):
    R = 1000
    return pl.pallas_call(
        _amm1_kernel,
        grid=(N // R,),
        in_specs=[
            pl.BlockSpec((R, 128), lambda i: (i, 0)),
            pl.BlockSpec((1, 128), lambda i: (0, 0)),
            pl.BlockSpec((1, 128), lambda i: (0, 0)),
        ],
        out_specs=[
            pl.BlockSpec((R, 16), lambda i: (i, 0)),
            pl.BlockSpec((R, 16), lambda i: (i, 0)),
        ],
        out_shape=[
            jax.ShapeDtypeStruct((NP, 16), _f32),
            jax.ShapeDtypeStruct((NP, 16), _f32),
        ],
    )(h1, att_src1, att_dst1)


def _epi0mm1_kernel(acc_ref, dnm_ref, bias_ref, g_ref, b_ref, w1_ref,
                    a1s_ref, a1d_ref, h1_ref, osrc_ref, odst_ref):
    d = dnm_ref[0] + dnm_ref[1]
    vs = []
    s1 = None
    s2 = None
    for h in range(8):
        v = acc_ref[h] / (d[:, h:h + 1] + 1e-16) + bias_ref[h]
        vs.append(v)
        t1 = jnp.sum(v, axis=1, keepdims=True)
        t2 = jnp.sum(v * v, axis=1, keepdims=True)
        s1 = t1 if s1 is None else s1 + t1
        s2 = t2 if s2 is None else s2 + t2
    mu = s1 * (1.0 / 1024.0)
    var = s2 * (1.0 / 1024.0) - mu * mu
    rstd = lax.rsqrt(var + 1e-5)
    h1 = None
    for h in range(8):
        yn = (vs[h] - mu) * rstd * g_ref[h] + b_ref[h]
        yn = jnp.maximum(yn, 0.0)
        t = jnp.dot(yn, w1_ref[h], preferred_element_type=_f32)
        h1 = t if h1 is None else h1 + t
    h1_ref[...] = h1
    onehot0 = (lax.broadcasted_iota(_i32, (1, 16), 1) == 0).astype(_f32)
    osrc_ref[...] = jnp.sum(h1 * a1s_ref[...], axis=1, keepdims=True) * onehot0
    odst_ref[...] = jnp.sum(h1 * a1d_ref[...], axis=1, keepdims=True) * onehot0


def _epi0mm1(accT, dnm, bias0, gamma0, beta0, W1r, att_src1, att_dst1):
    R = 400
    b = bias0.reshape(8, 1, 128)
    g = gamma0.reshape(8, 1, 128)
    bb = beta0.reshape(8, 1, 128)
    return pl.pallas_call(
        _epi0mm1_kernel,
        grid=(N // R,),
        in_specs=[
            pl.BlockSpec((8, R, 128), lambda i: (0, i, 0)),
            pl.BlockSpec((2, R, 16), lambda i: (0, i, 0)),
            pl.BlockSpec((8, 1, 128), lambda i: (0, 0, 0)),
            pl.BlockSpec((8, 1, 128), lambda i: (0, 0, 0)),
            pl.BlockSpec((8, 1, 128), lambda i: (0, 0, 0)),
            pl.BlockSpec((8, 128, 128), lambda i: (0, 0, 0)),
            pl.BlockSpec((1, 128), lambda i: (0, 0)),
            pl.BlockSpec((1, 128), lambda i: (0, 0)),
        ],
        out_specs=[
            pl.BlockSpec((R, 128), lambda i: (i, 0)),
            pl.BlockSpec((R, 16), lambda i: (i, 0)),
            pl.BlockSpec((R, 16), lambda i: (i, 0)),
        ],
        out_shape=[
            jax.ShapeDtypeStruct((N, 128), _f32),
            jax.ShapeDtypeStruct((NP, 16), _f32),
            jax.ShapeDtypeStruct((NP, 16), _f32),
        ],
    )(accT, dnm, b, g, bb, W1r, att_src1, att_dst1)


def _epi1_kernel(p_ref, dnm_ref, bias_ref, g_ref, b_ref, o_ref):
    d = dnm_ref[0] + dnm_ref[1]
    s = (p_ref[0] + p_ref[1]) / (d[:, 0:1] + 1e-16) + bias_ref[...]
    mu = jnp.mean(s, axis=1, keepdims=True)
    var = jnp.mean(s * s, axis=1, keepdims=True) - mu * mu
    o_ref[...] = (s - mu) * lax.rsqrt(var + 1e-5) * g_ref[...] + b_ref[...]


def _epi1(p, dnm, bias1, gamma1, beta1):
    R = 400
    return pl.pallas_call(
        _epi1_kernel,
        grid=(N // R,),
        in_specs=[
            pl.BlockSpec((2, R, 128), lambda i: (0, i, 0)),
            pl.BlockSpec((2, R, 16), lambda i: (0, i, 0)),
            pl.BlockSpec((1, 128), lambda i: (0, 0)),
            pl.BlockSpec((1, 128), lambda i: (0, 0)),
            pl.BlockSpec((1, 128), lambda i: (0, 0)),
        ],
        out_specs=pl.BlockSpec((R, 128), lambda i: (i, 0)),
        out_shape=jax.ShapeDtypeStruct((N, 128), _f32),
    )(p, dnm, bias1.reshape(1, 128), gamma1.reshape(1, 128),
      beta1.reshape(1, 128))


def _zero_vmem(zb, rows, cols):
    zv = jnp.zeros((LANES,), _f32)

    def body(i, _):
        for j in range(cols // LANES):
            zb[i, pl.ds(j * LANES, LANES)] = zv
        return 0

    lax.fori_loop(0, rows, body, 0)


def _make_pass_a():
    mesh = plsc.VectorSubcoreMesh(core_axis_name="c", subcore_axis_name="s")
    e_per_worker = E // (NC * NS)
    nchunks = e_per_worker // CA
    NBUF = 4
    nstages = -(-(nchunks + 2) // NBUF) * NBUF
    nsteps = nstages // NBUF

    gissued = [0] * NBUF
    gwaited = [0] * NBUF
    for c in (0, 1):
        gissued[c % NBUF] += 1
    for m in range(nstages):
        gissued[(m + 2) % NBUF] += 1
        if m < nchunks:
            gwaited[m % NBUF] += 1

    scratch = []
    for _ in range(NBUF):
        scratch += [
            pltpu.VMEM((CA,), _i32),
            pltpu.VMEM((CA,), _i32),
            pltpu.VMEM((CA, 16), _f32),
            pltpu.VMEM((CA, 16), _f32),
            pltpu.VMEM((CA, 16), _f32),
        ]
    scratch += [pltpu.VMEM_SHARED((NP, 16), _f32)]
    scratch += [pltpu.SemaphoreType.DMA] * (4 * NBUF)

    @functools.partial(
        pl.kernel,
        out_type=[
            jax.ShapeDtypeStruct((E, 16), _f32),
            jax.ShapeDtypeStruct((NC, NP, 16), _f32),
        ],
        mesh=mesh,
        compiler_params=pltpu.CompilerParams(use_tc_tiling_on_sc=False),
        scratch_types=scratch,
    )
    def pass_a(srce_ref, dste_ref, asrc_ref, adst_ref, zeros_ref,
               exr_ref, dnm_ref, *scr):
        src_b = [scr[5 * b + 0] for b in range(NBUF)]
        dst_b = [scr[5 * b + 1] for b in range(NBUF)]
        ga_b = [scr[5 * b + 2] for b in range(NBUF)]
        gb_b = [scr[5 * b + 3] for b in range(NBUF)]
        ex_b = [scr[5 * b + 4] for b in range(NBUF)]
        dsh = scr[5 * NBUF]
        gsem = scr[5 * NBUF + 1:5 * NBUF + 1 + NBUF]
        ssem = scr[5 * NBUF + 1 + NBUF:5 * NBUF + 1 + 2 * NBUF]
        wsem = scr[5 * NBUF + 1 + 2 * NBUF:5 * NBUF + 1 + 3 * NBUF]
        isem = scr[5 * NBUF + 1 + 3 * NBUF:5 * NBUF + 1 + 4 * NBUF]

        core = lax.axis_index("c")
        sub = lax.axis_index("s")
        base = core * (E // NC) + sub * e_per_worker

        for z in range(ROWS_PER_SUB // ZROWS):
            pltpu.sync_copy(
                zeros_ref,
                dsh.at[pl.ds(sub * ROWS_PER_SUB + z * ZROWS, ZROWS), :])
        plsc.subcore_barrier()

        def idx_issue(b, cc, sync):
            off = base + cc * CA
            if sync:
                pltpu.sync_copy(srce_ref.at[pl.ds(off, CA)], src_b[b])
                pltpu.sync_copy(dste_ref.at[pl.ds(off, CA)], dst_b[b])
            else:
                pltpu.async_copy(srce_ref.at[pl.ds(off, CA)], src_b[b],
                                 isem[b])
                pltpu.async_copy(dste_ref.at[pl.ds(off, CA)], dst_b[b],
                                 isem[b])

        def idx_wait(b):
            pltpu.make_async_copy(srce_ref.at[pl.ds(0, CA)], src_b[b],
                                  isem[b]).wait()
            pltpu.make_async_copy(dste_ref.at[pl.ds(0, CA)], dst_b[b],
                                  isem[b]).wait()

        def gather_issue(b):
            pltpu.async_copy(asrc_ref.at[src_b[b]], ga_b[b], gsem[b])
            pltpu.async_copy(adst_ref.at[dst_b[b]], gb_b[b], gsem[b])

        def gather_wait(b):
            pltpu.make_async_copy(asrc_ref.at[src_b[b]], ga_b[b],
                                  gsem[b]).wait()
            pltpu.make_async_copy(adst_ref.at[dst_b[b]], gb_b[b],
                                  gsem[b]).wait()

        idx_issue(0, jnp.int32(0), True)
        gather_issue(0)
        idx_issue(1, jnp.int32(1), True)
        gather_issue(1)

        def tloop(t, _):
            for j in range(NBUF):
                m = t * NBUF + j
                b2 = (j + 2) % NBUF
                bm = j

                @pl.when(jnp.logical_and(m >= 2, m - 2 < nchunks))
                def _():
                    pltpu.make_async_copy(
                        ex_b[b2], dsh.at[dst_b[b2]], ssem[b2]).wait()
                    pltpu.make_async_copy(
                        ex_b[b2], exr_ref.at[pl.ds(0, CA), :],
                        wsem[b2]).wait()

                idx_issue(b2, jnp.minimum(m + 2, nchunks - 1), False)

                @pl.when(m < nchunks)
                def _():
                    gather_wait(bm)
                    ga = ga_b[bm]
                    gb = gb_b[bm]
                    ex = ex_b[bm]

                    def comp(g, _):
                        for jj in range(4):
                            i = g * 4 + jj
                            v = ga[i, :] + gb[i, :]
                            v = jnp.where(v >= 0.0, v, 0.2 * v)
                            ex[i, :] = jnp.exp(v)
                        return 0

                    lax.fori_loop(0, CA // 4, comp, 0)
                    pltpu.async_copy(ex, dsh.at[dst_b[bm]], ssem[bm],
                                     add=True)
                    off = base + m * CA
                    pltpu.async_copy(ex, exr_ref.at[pl.ds(off, CA), :],
                                     wsem[bm])

                idx_wait(b2)
                gather_issue(b2)
            return 0

        lax.fori_loop(0, nsteps, tloop, 0)

        for b in range(NBUF):
            for _ in range(gissued[b] - gwaited[b]):
                gather_wait(b)
        plsc.subcore_barrier()

        pltpu.sync_copy(
            dsh.at[pl.ds(sub * ROWS_PER_SUB, ROWS_PER_SUB), :],
            dnm_ref.at[core, pl.ds(sub * ROWS_PER_SUB, ROWS_PER_SUB), :])

    return pass_a


def _make_pass_b(split_heads):
    mesh = plsc.VectorSubcoreMesh(core_axis_name="c", subcore_axis_name="s")
    if split_heads:
        hpc = 4
        e_per_worker = E // NS
        out_major = 8
    else:
        hpc = 1
        e_per_worker = E // (NC * NS)
        out_major = NC
    nchunks = e_per_worker // CB
    NBUF = 4
    nstages = -(-(nchunks + 2) // NBUF) * NBUF
    nsteps = nstages // NBUF

    gissued = [0] * NBUF
    gwaited = [0] * NBUF
    for c in (0, 1):
        gissued[c % NBUF] += 1
    for m in range(nstages):
        gissued[(m + 2) % NBUF] += 1
        if m < nchunks:
            gwaited[m % NBUF] += 1

    scratch = []
    for _ in range(NBUF):
        scratch += [
            pltpu.VMEM((CB,), _i32),
            pltpu.VMEM((CB,), _i32),
            pltpu.VMEM((CB,), _i32),
            pltpu.VMEM((CB, 16), _f32),
            pltpu.VMEM((CB, 128), _f32),
        ]
    scratch += [pltpu.VMEM_SHARED((NP, 128), _f32)]
    scratch += [pltpu.SemaphoreType.DMA] * (3 * NBUF)

    @functools.partial(
        pl.kernel,
        out_type=jax.ShapeDtypeStruct((out_major, NP, 128), _f32),
        mesh=mesh,
        compiler_params=pltpu.CompilerParams(use_tc_tiling_on_sc=False),
        scratch_types=scratch,
    )
    def pass_b(h_ref, ext_ref, srce_ref, dste_ref, zeros_ref, out_ref, *scr):
        src_b = [scr[5 * b + 0] for b in range(NBUF)]
        dst_b = [scr[5 * b + 1] for b in range(NBUF)]
        gidx_b = [scr[5 * b + 2] for b in range(NBUF)]
        ex_b = [scr[5 * b + 3] for b in range(NBUF)]
        rows_b = [scr[5 * b + 4] for b in range(NBUF)]
        ash = scr[5 * NBUF]
        gsem = scr[5 * NBUF + 1:5 * NBUF + 1 + NBUF]
        ssem = scr[5 * NBUF + 1 + NBUF:5 * NBUF + 1 + 2 * NBUF]
        isem = scr[5 * NBUF + 1 + 2 * NBUF:5 * NBUF + 1 + 3 * NBUF]

        core = lax.axis_index("c")
        sub = lax.axis_index("s")

        def idx_issue(b, cc, sync):
            off = ebase[0] + cc * CB
            if sync:
                pltpu.sync_copy(srce_ref.at[pl.ds(off, CB)], src_b[b])
                pltpu.sync_copy(dste_ref.at[pl.ds(off, CB)], dst_b[b])
                pltpu.sync_copy(ext_ref.at[pl.ds(off, CB), :], ex_b[b])
            else:
                pltpu.async_copy(srce_ref.at[pl.ds(off, CB)], src_b[b],
                                 isem[b])
                pltpu.async_copy(dste_ref.at[pl.ds(off, CB)], dst_b[b],
                                 isem[b])
                pltpu.async_copy(ext_ref.at[pl.ds(off, CB), :], ex_b[b],
                                 isem[b])

        def idx_wait(b):
            pltpu.make_async_copy(srce_ref.at[pl.ds(0, CB)], src_b[b],
                                  isem[b]).wait()
            pltpu.make_async_copy(dste_ref.at[pl.ds(0, CB)], dst_b[b],
                                  isem[b]).wait()
            pltpu.make_async_copy(ext_ref.at[pl.ds(0, CB), :], ex_b[b],
                                  isem[b]).wait()

        def gather_issue(b, hoff):
            for g in range(CB // LANES):
                sl = pl.ds(g * LANES, LANES)
                gidx_b[b][sl] = src_b[b][sl] + hoff
            pltpu.async_copy(h_ref.at[gidx_b[b]], rows_b[b], gsem[b])

        ebase = [jnp.int32(0)]

        def one_head(hl, _):
            if split_heads:
                h = core * hpc + hl
                ebase[0] = sub * e_per_worker
            else:
                h = hl
                ebase[0] = core * (E // NC) + sub * e_per_worker
            hvec = jnp.zeros((LANES,), _i32) + h
            hoff = h * N

            for z in range(ROWS_PER_SUB // ZROWS):
                pltpu.sync_copy(
                    zeros_ref,
                    ash.at[pl.ds(sub * ROWS_PER_SUB + z * ZROWS, ZROWS), :])
            plsc.subcore_barrier()

            idx_issue(0, jnp.int32(0), True)
            gather_issue(0, hoff)
            idx_issue(1, jnp.int32(1), True)
            gather_issue(1, hoff)

            def tloop(t, _):
                for j in range(NBUF):
                    m = t * NBUF + j
                    b2 = (j + 2) % NBUF
                    bm = j

                    @pl.when(jnp.logical_and(m >= 2, m - 2 < nchunks))
                    def _():
                        pltpu.make_async_copy(
                            rows_b[b2], ash.at[dst_b[b2]], ssem[b2]).wait()

                    idx_issue(b2, jnp.minimum(m + 2, nchunks - 1), False)

                    @pl.when(m < nchunks)
                    def _():
                        pltpu.make_async_copy(
                            h_ref.at[gidx_b[bm]], rows_b[bm],
                            gsem[bm]).wait()
                        rv = rows_b[bm]
                        ev = ex_b[bm]

                        def scale(g, _):
                            for jj in range(4):
                                i = g * 4 + jj
                                w = ev[i, :][hvec]
                                for q in range(128 // LANES):
                                    sl = pl.ds(q * LANES, LANES)
                                    rv[i, sl] = rv[i, sl] * w
                            return 0

                        lax.fori_loop(0, CB // 4, scale, 0)
                        pltpu.async_copy(rv, ash.at[dst_b[bm]], ssem[bm],
                                         add=True)

                    idx_wait(b2)
                    gather_issue(b2, hoff)
                return 0

            lax.fori_loop(0, nsteps, tloop, 0)

            for b in range(NBUF):
                for _ in range(gissued[b] - gwaited[b]):
                    pltpu.make_async_copy(
                        h_ref.at[gidx_b[b]], rows_b[b], gsem[b]).wait()
            plsc.subcore_barrier()
            dst_major = h if split_heads else core
            pltpu.sync_copy(
                ash.at[pl.ds(sub * ROWS_PER_SUB, ROWS_PER_SUB), :],
                out_ref.at[dst_major,
                           pl.ds(sub * ROWS_PER_SUB, ROWS_PER_SUB), :])
            plsc.subcore_barrier()
            return 0

        lax.fori_loop(0, hpc, one_head, 0)

    return pass_b


_pass_a = _make_pass_a()
_pass_b0 = _make_pass_b(True)
_pass_b1 = _make_pass_b(False)


def kernel(x, edge_index, W0, att_src0, att_dst0, bias0, gamma0, beta0,
           W1, att_src1, att_dst1, bias1, gamma1, beta1):
    hT, asrc0, adst0 = _mm0(x, W0, att_src0, att_dst0)
    src = edge_index[0]
    dst = edge_index[1]
    zeros = jnp.zeros((ZROWS, 128), _f32)
    zeros16 = jnp.zeros((ZROWS, 16), _f32)
    exr0, dnm0 = _pass_a(src, dst, asrc0, adst0, zeros16)
    accT = _pass_b0(hT.reshape(8 * N, 128), exr0, src, dst, zeros)
    h1, asrc1, adst1 = _epi0mm1(accT, dnm0, bias0, gamma0, beta0,
                                W1.reshape(8, 128, 128), att_src1, att_dst1)
    exr1, dnm1 = _pass_a(src, dst, asrc1, adst1, zeros16)
    p = _pass_b1(h1, exr1, src, dst, zeros)
    out = _epi1(p, dnm1, bias1, gamma1, beta1)
    return out
